# baseline (speedup 1.0000x reference)
import functools

import jax
import jax.numpy as jnp
from jax import lax
from jax.experimental import pallas as pl
from jax.experimental.pallas import tpu as pltpu
from jax.experimental.pallas import tpu_sc as plsc

_NUM_REVIEWER = 100000
_NUM_DINER = 1000000
_EMB = 64
_BATCH = 16384
_NDIN = 5
_NC = 2
_NS = 16
_NW = _NC * _NS
_PER_W = _BATCH // _NW
_CHUNK = 128
_NCHUNK = _PER_W // _CHUNK
_L = 16


def _log1p_of(u):
    z = u / (u + 2.0)
    z2 = z * z
    p = 1.0 + z2 * (1.0 / 3.0 + z2 * (1.0 / 5.0 + z2 * (1.0 / 7.0 + z2 * (1.0 / 9.0))))
    return 2.0 * z * p


def _softplus(x):
    return jnp.maximum(x, 0.0) + _log1p_of(jnp.exp(-jnp.abs(x)))


def _body(rev_hbm, din_hbm, w_hbm, ridx_hbm, didx_hbm, out_hbm,
          ridx_v, didx_v, w_v, rev_rows, din_rows, loss_v, sem):
    wid = lax.axis_index("s") * _NC + lax.axis_index("c")
    iota = lax.iota(jnp.int32, _L)

    loss = jnp.zeros((_L,), jnp.float32)
    for c in range(_NCHUNK):
        base = wid * _PER_W + c * _CHUNK
        pltpu.sync_copy(ridx_hbm.at[pl.ds(base, _CHUNK)], ridx_v)
        for j in range(_NDIN):
            pltpu.sync_copy(didx_hbm.at[pl.ds(base * _NDIN + j * _CHUNK, _CHUNK)],
                            didx_v.at[j])
        pltpu.sync_copy(w_hbm.at[pl.ds(base, _CHUNK)], w_v)
        copies = [pltpu.async_copy(rev_hbm.at[ridx_v], rev_rows, sem)]
        for j in range(_NDIN):
            copies.append(pltpu.async_copy(
                din_hbm.at[didx_v.at[j]],
                din_rows.at[pl.ds(j * _CHUNK, _CHUNK)], sem))
        for cp in copies:
            cp.wait()

        def group_step(g, loss_acc):
            samp = g * _L + iota
            dbase = samp * _NDIN
            w = w_v[pl.ds(g * _L, _L)]

            def e_step(e, accs):
                col = jnp.broadcast_to(e, (_L,))
                rcol = plsc.load_gather(rev_rows, [samp, col])
                new = []
                for d in range(_NDIN):
                    dcol = plsc.load_gather(din_rows, [dbase + d, col])
                    new.append(accs[d] + rcol * dcol)
                return tuple(new)

            accs = lax.fori_loop(
                0, _EMB, e_step,
                tuple(jnp.zeros((_L,), jnp.float32) for _ in range(_NDIN)))
            contrib = jnp.zeros((_L,), jnp.float32)
            for d in range(1, _NDIN):
                contrib = contrib + _softplus(accs[d] - accs[0])
            return loss_acc + w * contrib

        loss = lax.fori_loop(0, _CHUNK // _L, group_step, loss)

    loss_v[...] = loss
    pltpu.sync_copy(loss_v, out_hbm.at[wid])


@jax.jit
def _run(rev_emb, din_emb, weights, reviewers, diners):
    mesh = plsc.VectorSubcoreMesh(core_axis_name="c", subcore_axis_name="s")
    partials = pl.kernel(
        _body,
        out_type=jax.ShapeDtypeStruct((_NW, _L), jnp.float32),
        mesh=mesh,
        scratch_types=[
            pltpu.VMEM((_CHUNK,), jnp.int32),
            pltpu.VMEM((_NDIN, _CHUNK), jnp.int32),
            pltpu.VMEM((_CHUNK,), jnp.float32),
            pltpu.VMEM((_CHUNK, _EMB), jnp.float32),
            pltpu.VMEM((_CHUNK * _NDIN, _EMB), jnp.float32),
            pltpu.VMEM((_L,), jnp.float32),
            pltpu.SemaphoreType.DMA,
        ],
        compiler_params=pltpu.CompilerParams(
            needs_layout_passes=False, use_tc_tiling_on_sc=False),
    )(rev_emb, din_emb, weights, reviewers, diners)
    return jnp.sum(partials) * (1.0 / (_BATCH * (_NDIN - 1)))


def kernel(reviewer_emb, diner_emb, weights, reviewers, diners):
    return _run(
        reviewer_emb,
        diner_emb,
        jnp.reshape(weights, (_BATCH,)),
        reviewers.astype(jnp.int32),
        jnp.reshape(diners.astype(jnp.int32), (_BATCH * _NDIN,)),
    )
